# baseline (speedup 1.0000x reference)
import functools

import jax
import jax.numpy as jnp
from jax import lax
from jax.experimental import pallas as pl
from jax.experimental.pallas import tpu as pltpu
from jax.experimental.pallas import tpu_sc as plsc

BATCH = 16384
VOCAB = 1000000
EMB = 64
HIDDEN = 32

BLKV = 8192
NVBLK = (VOCAB + BLKV - 1) // BLKV
VOCABP = NVBLK * BLKV

NC = 2
NS = 16
NW = NC * NS
B_PER_W = BATCH // NW


def _embed_matmul_body(et1_ref, et2_ref, w1a_ref, w1b_ref, h_ref):
    dn = (((0,), (0,)), ((), ()))
    et1 = et1_ref[...].astype(jnp.bfloat16)
    et2 = et2_ref[...].astype(jnp.bfloat16)
    w1a = w1a_ref[...].astype(jnp.bfloat16)
    w1b = w1b_ref[...].astype(jnp.bfloat16)
    h1 = lax.dot_general(et1, w1a, dn, preferred_element_type=jnp.float32)
    h2 = lax.dot_general(et2, w1b, dn, preferred_element_type=jnp.float32)
    h_ref[...] = jnp.concatenate([h1, h2, h1, h2], axis=1).astype(jnp.bfloat16)


def _embed_matmul(ET1, ET2, W1a, W1b):
    return pl.pallas_call(
        _embed_matmul_body,
        grid=(NVBLK,),
        in_specs=[
            pl.BlockSpec((EMB, BLKV), lambda i: (0, i)),
            pl.BlockSpec((EMB, BLKV), lambda i: (0, i)),
            pl.BlockSpec((EMB, HIDDEN), lambda i: (0, 0)),
            pl.BlockSpec((EMB, HIDDEN), lambda i: (0, 0)),
        ],
        out_specs=pl.BlockSpec((BLKV, 128), lambda i: (i, 0)),
        out_shape=jax.ShapeDtypeStruct((VOCABP, 128), jnp.bfloat16),
        compiler_params=pltpu.CompilerParams(
            dimension_semantics=("parallel",),
            fuse_transposed_lhs_in_matmul=True,
        ),
    )(ET1, ET2, W1a, W1b)


def _sc_gather_one(H, idx):
    mesh = plsc.VectorSubcoreMesh(core_axis_name="c", subcore_axis_name="s")

    @functools.partial(
        pl.kernel,
        mesh=mesh,
        out_type=jax.ShapeDtypeStruct((BATCH, 128), jnp.bfloat16),
        scratch_types=[
            pltpu.VMEM((B_PER_W,), jnp.int32),
            pltpu.VMEM((B_PER_W, 128), jnp.bfloat16),
            pltpu.SemaphoreType.DMA,
        ],
        compiler_params=pltpu.CompilerParams(use_tc_tiling_on_sc=False),
    )
    def k(h_hbm, i_hbm, o_hbm, i_v, r_v, s):
        wid = lax.axis_index("s") * NC + lax.axis_index("c")
        base = wid * B_PER_W
        pltpu.sync_copy(i_hbm.at[pl.ds(base, B_PER_W)], i_v)
        pltpu.async_copy(h_hbm.at[i_v], r_v, s).wait()
        pltpu.sync_copy(r_v, o_hbm.at[pl.ds(base, B_PER_W)])

    return k(H, idx)


def _mlp_body(g1_ref, g2_ref, b1_ref, w2_ref, b2_ref, o_ref):
    g1 = g1_ref[:, :HIDDEN].astype(jnp.float32)
    g2 = g2_ref[:, HIDDEN:2 * HIDDEN].astype(jnp.float32)
    h = jnp.maximum(g1 + g2 + b1_ref[...], 0.0)
    o = jnp.dot(h, w2_ref[...], preferred_element_type=jnp.float32) + b2_ref[...]
    o_ref[...] = jax.nn.sigmoid(o)


def _tc_mlp(g1, g2, b1, W2, b2):
    BLK = 4096
    return pl.pallas_call(
        _mlp_body,
        grid=(BATCH // BLK,),
        in_specs=[
            pl.BlockSpec((BLK, 128), lambda i: (i, 0)),
            pl.BlockSpec((BLK, 128), lambda i: (i, 0)),
            pl.BlockSpec((1, HIDDEN), lambda i: (0, 0)),
            pl.BlockSpec((HIDDEN, 1), lambda i: (0, 0)),
            pl.BlockSpec((1, 1), lambda i: (0, 0)),
        ],
        out_specs=pl.BlockSpec((BLK, 1), lambda i: (i, 0)),
        out_shape=jax.ShapeDtypeStruct((BATCH, 1), jnp.float32),
    )(g1, g2, b1, W2, b2)


def kernel(inputs, E1, E2, W1, b1, W2, b2):
    idx1 = inputs[:, 0]
    idx2 = inputs[:, 1]
    ET1 = E1.T
    ET2 = E2.T
    W1a = W1[:EMB]
    W1b = W1[EMB:]
    H = _embed_matmul(ET1, ET2, W1a, W1b)
    g1 = _sc_gather_one(H, idx1)
    g2 = _sc_gather_one(H, idx2)
    return _tc_mlp(g1, g2, b1.reshape(1, HIDDEN), W2, b2.reshape(1, 1))


# speedup vs baseline: 2.8374x; 2.8374x over previous
import functools

import jax
import jax.numpy as jnp
from jax import lax
from jax.experimental import pallas as pl
from jax.experimental.pallas import tpu as pltpu
from jax.experimental.pallas import tpu_sc as plsc

BATCH = 16384
VOCAB = 1000000
EMB = 64
HIDDEN = 32

BLKV = 16384
HALFV = BLKV // 2
NVBLK = (VOCAB + BLKV - 1) // BLKV
NROWS = NVBLK * HALFV

NC = 2
NS = 16
NW = NC * NS
B_PER_W = BATCH // NW


def _embed_matmul_body(et1_ref, et2_ref, w1a_ref, w1b_ref, h_ref):
    dn = (((0,), (0,)), ((), ()))
    et1 = et1_ref[...].astype(jnp.bfloat16)
    et2 = et2_ref[...].astype(jnp.bfloat16)
    w1a = w1a_ref[...].astype(jnp.bfloat16)
    w1b = w1b_ref[...].astype(jnp.bfloat16)
    h1 = lax.dot_general(et1, w1a, dn, preferred_element_type=jnp.float32)
    h2 = lax.dot_general(et2, w1b, dn, preferred_element_type=jnp.float32)
    def pack(h):
        ulo = lax.bitcast_convert_type(h[:HALFV], jnp.uint32)
        uhi = lax.bitcast_convert_type(h[HALFV:], jnp.uint32)
        return (uhi & jnp.uint32(0xFFFF0000)) | (ulo >> 16)

    w1 = pack(h1)
    w2 = pack(h2)
    w = jnp.concatenate([w1, w2, w1, w2], axis=1)
    h_ref[...] = lax.bitcast_convert_type(w, jnp.float32)


def _embed_matmul(ET1, ET2, W1a, W1b):
    return pl.pallas_call(
        _embed_matmul_body,
        grid=(NVBLK,),
        in_specs=[
            pl.BlockSpec((EMB, BLKV), lambda i: (0, i)),
            pl.BlockSpec((EMB, BLKV), lambda i: (0, i)),
            pl.BlockSpec((EMB, HIDDEN), lambda i: (0, 0)),
            pl.BlockSpec((EMB, HIDDEN), lambda i: (0, 0)),
        ],
        out_specs=pl.BlockSpec((HALFV, 128), lambda i: (i, 0)),
        out_shape=jax.ShapeDtypeStruct((NROWS, 128), jnp.float32),
        compiler_params=pltpu.CompilerParams(
            dimension_semantics=("parallel",),
        ),
    )(ET1, ET2, W1a, W1b)


def _sc_gather_one(H, row):
    mesh = plsc.VectorSubcoreMesh(core_axis_name="c", subcore_axis_name="s")

    @functools.partial(
        pl.kernel,
        mesh=mesh,
        out_type=jax.ShapeDtypeStruct((BATCH, 128), jnp.float32),
        scratch_types=[
            pltpu.VMEM((B_PER_W,), jnp.int32),
            pltpu.VMEM((B_PER_W, 128), jnp.float32),
            pltpu.SemaphoreType.DMA,
        ],
        compiler_params=pltpu.CompilerParams(use_tc_tiling_on_sc=False),
    )
    def k(h_hbm, i_hbm, o_hbm, i_v, r_v, s):
        wid = lax.axis_index("s") * NC + lax.axis_index("c")
        base = wid * B_PER_W
        pltpu.sync_copy(i_hbm.at[pl.ds(base, B_PER_W)], i_v)
        pltpu.async_copy(h_hbm.at[i_v], r_v, s).wait()
        pltpu.sync_copy(r_v, o_hbm.at[pl.ds(base, B_PER_W)])

    return k(H, row)


def _unpack_half(g, half, lane0):
    bits = lax.bitcast_convert_type(g[:, lane0:lane0 + HIDDEN], jnp.uint32)
    lo = lax.bitcast_convert_type(bits << 16, jnp.float32)
    hi = lax.bitcast_convert_type(bits & jnp.uint32(0xFFFF0000), jnp.float32)
    return jnp.where(half > 0, hi, lo)


def _mlp_body(g1_ref, g2_ref, f1_ref, f2_ref, b1_ref, w2_ref, b2_ref, o_ref):
    h1 = _unpack_half(g1_ref[...], f1_ref[...], 0)
    h2 = _unpack_half(g2_ref[...], f2_ref[...], HIDDEN)
    h = jnp.maximum(h1 + h2 + b1_ref[...], 0.0)
    o = jnp.dot(h, w2_ref[...], preferred_element_type=jnp.float32) + b2_ref[...]
    o_ref[...] = jax.nn.sigmoid(o)


def _tc_mlp(g1, g2, f1, f2, b1, W2, b2):
    BLK = 4096
    return pl.pallas_call(
        _mlp_body,
        grid=(BATCH // BLK,),
        in_specs=[
            pl.BlockSpec((BLK, 128), lambda i: (i, 0)),
            pl.BlockSpec((BLK, 128), lambda i: (i, 0)),
            pl.BlockSpec((BLK, 1), lambda i: (i, 0)),
            pl.BlockSpec((BLK, 1), lambda i: (i, 0)),
            pl.BlockSpec((1, HIDDEN), lambda i: (0, 0)),
            pl.BlockSpec((HIDDEN, 1), lambda i: (0, 0)),
            pl.BlockSpec((1, 1), lambda i: (0, 0)),
        ],
        out_specs=pl.BlockSpec((BLK, 1), lambda i: (i, 0)),
        out_shape=jax.ShapeDtypeStruct((BATCH, 1), jnp.float32),
    )(g1, g2, f1, f2, b1, W2, b2)


_BLKV_BITS = BLKV.bit_length() - 1
_HALFV_BITS = HALFV.bit_length() - 1


def _row_half(idx):
    blk = idx >> _BLKV_BITS
    r = idx & (BLKV - 1)
    row = (blk << _HALFV_BITS) | (r & (HALFV - 1))
    half = r >> _HALFV_BITS
    return row, half


def kernel(inputs, E1, E2, W1, b1, W2, b2):
    idx1 = inputs[:, 0]
    idx2 = inputs[:, 1]
    ET1 = E1.T
    ET2 = E2.T
    W1a = W1[:EMB]
    W1b = W1[EMB:]
    H = _embed_matmul(ET1, ET2, W1a, W1b)
    row1, half1 = _row_half(idx1)
    row2, half2 = _row_half(idx2)
    g1 = _sc_gather_one(H, row1)
    g2 = _sc_gather_one(H, row2)
    return _tc_mlp(g1, g2, half1.reshape(BATCH, 1), half2.reshape(BATCH, 1),
                   b1.reshape(1, HIDDEN), W2, b2.reshape(1, 1))


# speedup vs baseline: 3.5662x; 1.2569x over previous
import functools

import jax
import jax.numpy as jnp
from jax import lax
from jax.experimental import pallas as pl
from jax.experimental.pallas import tpu as pltpu
from jax.experimental.pallas import tpu_sc as plsc

BATCH = 16384
VOCAB = 1000000
EMB = 64
HIDDEN = 32

BLKV = 16384
QV = BLKV // 4
NVBLK = (VOCAB + BLKV - 1) // BLKV
NROWS = NVBLK * QV

NC = 2
NS = 16
NW = NC * NS
B_PER_W = BATCH // NW


def _embed_matmul_body(et1_ref, et2_ref, w1a_ref, w1b_ref, h_ref):
    dn = (((0,), (0,)), ((), ()))
    et1 = et1_ref[...].astype(jnp.bfloat16)
    et2 = et2_ref[...].astype(jnp.bfloat16)
    w1a = w1a_ref[...].astype(jnp.bfloat16)
    w1b = w1b_ref[...].astype(jnp.bfloat16)
    h1 = lax.dot_general(et1, w1a, dn, preferred_element_type=jnp.float32)
    h2 = lax.dot_general(et2, w1b, dn, preferred_element_type=jnp.float32)
    def pack(lo, hi):
        ulo = lax.bitcast_convert_type(lo, jnp.uint32)
        uhi = lax.bitcast_convert_type(hi, jnp.uint32)
        return (uhi & jnp.uint32(0xFFFF0000)) | (ulo >> 16)

    w = jnp.concatenate([
        pack(h1[:QV], h1[QV:2 * QV]),
        pack(h2[:QV], h2[QV:2 * QV]),
        pack(h1[2 * QV:3 * QV], h1[3 * QV:]),
        pack(h2[2 * QV:3 * QV], h2[3 * QV:]),
    ], axis=1)
    h_ref[...] = lax.bitcast_convert_type(w, jnp.float32)


def _embed_matmul(ET1, ET2, W1a, W1b):
    return pl.pallas_call(
        _embed_matmul_body,
        grid=(NVBLK,),
        in_specs=[
            pl.BlockSpec((EMB, BLKV), lambda i: (0, i)),
            pl.BlockSpec((EMB, BLKV), lambda i: (0, i)),
            pl.BlockSpec((EMB, HIDDEN), lambda i: (0, 0)),
            pl.BlockSpec((EMB, HIDDEN), lambda i: (0, 0)),
        ],
        out_specs=pl.BlockSpec((QV, 128), lambda i: (i, 0)),
        out_shape=jax.ShapeDtypeStruct((NROWS, 128), jnp.float32),
        compiler_params=pltpu.CompilerParams(
            dimension_semantics=("parallel",),
        ),
    )(ET1, ET2, W1a, W1b)


def _sc_gather_one(H, row):
    mesh = plsc.VectorSubcoreMesh(core_axis_name="c", subcore_axis_name="s")

    @functools.partial(
        pl.kernel,
        mesh=mesh,
        out_type=jax.ShapeDtypeStruct((BATCH, 128), jnp.float32),
        scratch_types=[
            pltpu.VMEM((B_PER_W,), jnp.int32),
            pltpu.VMEM((B_PER_W, 128), jnp.float32),
            pltpu.SemaphoreType.DMA,
        ],
        compiler_params=pltpu.CompilerParams(use_tc_tiling_on_sc=False),
    )
    def k(h_hbm, i_hbm, o_hbm, i_v, r_v, s):
        wid = lax.axis_index("s") * NC + lax.axis_index("c")
        base = wid * B_PER_W
        pltpu.sync_copy(i_hbm.at[pl.ds(base, B_PER_W)], i_v)
        pltpu.async_copy(h_hbm.at[i_v], r_v, s).wait()
        pltpu.sync_copy(r_v, o_hbm.at[pl.ds(base, B_PER_W)])

    return k(H, row)


def _unpack_half(g, half, lane0):
    bits = lax.bitcast_convert_type(g[:, lane0:lane0 + HIDDEN], jnp.uint32)
    lo = lax.bitcast_convert_type(bits << 16, jnp.float32)
    hi = lax.bitcast_convert_type(bits & jnp.uint32(0xFFFF0000), jnp.float32)
    return jnp.where(half > 0, hi, lo)


def _select_quarter(g, sel, lane0):
    a = _unpack_half(g, sel & 1, lane0)
    b = _unpack_half(g, sel & 1, lane0 + 2 * HIDDEN)
    return jnp.where((sel >> 1) > 0, b, a)


def _mlp_body(g1_ref, g2_ref, f1_ref, f2_ref, b1_ref, w2_ref, b2_ref, o_ref):
    h1 = _select_quarter(g1_ref[...], f1_ref[...], 0)
    h2 = _select_quarter(g2_ref[...], f2_ref[...], HIDDEN)
    h = jnp.maximum(h1 + h2 + b1_ref[...], 0.0)
    o = jnp.dot(h, w2_ref[...], preferred_element_type=jnp.float32) + b2_ref[...]
    o_ref[...] = jax.nn.sigmoid(o)


def _tc_mlp(g1, g2, f1, f2, b1, W2, b2):
    BLK = 4096
    return pl.pallas_call(
        _mlp_body,
        grid=(BATCH // BLK,),
        in_specs=[
            pl.BlockSpec((BLK, 128), lambda i: (i, 0)),
            pl.BlockSpec((BLK, 128), lambda i: (i, 0)),
            pl.BlockSpec((BLK, 1), lambda i: (i, 0)),
            pl.BlockSpec((BLK, 1), lambda i: (i, 0)),
            pl.BlockSpec((1, HIDDEN), lambda i: (0, 0)),
            pl.BlockSpec((HIDDEN, 1), lambda i: (0, 0)),
            pl.BlockSpec((1, 1), lambda i: (0, 0)),
        ],
        out_specs=pl.BlockSpec((BLK, 1), lambda i: (i, 0)),
        out_shape=jax.ShapeDtypeStruct((BATCH, 1), jnp.float32),
    )(g1, g2, f1, f2, b1, W2, b2)


_BLKV_BITS = BLKV.bit_length() - 1
_QV_BITS = QV.bit_length() - 1


def _row_half(idx):
    blk = idx >> _BLKV_BITS
    r = idx & (BLKV - 1)
    row = (blk << _QV_BITS) | (r & (QV - 1))
    sel = r >> _QV_BITS
    return row, sel


def kernel(inputs, E1, E2, W1, b1, W2, b2):
    idx1 = inputs[:, 0]
    idx2 = inputs[:, 1]
    ET1 = E1.T
    ET2 = E2.T
    W1a = W1[:EMB]
    W1b = W1[EMB:]
    H = _embed_matmul(ET1, ET2, W1a, W1b)
    row1, half1 = _row_half(idx1)
    row2, half2 = _row_half(idx2)
    g1 = _sc_gather_one(H, row1)
    g2 = _sc_gather_one(H, row2)
    return _tc_mlp(g1, g2, half1.reshape(BATCH, 1), half2.reshape(BATCH, 1),
                   b1.reshape(1, HIDDEN), W2, b2.reshape(1, 1))


# speedup vs baseline: 3.6610x; 1.0266x over previous
import functools

import jax
import jax.numpy as jnp
from jax import lax
from jax.experimental import pallas as pl
from jax.experimental.pallas import tpu as pltpu
from jax.experimental.pallas import tpu_sc as plsc

BATCH = 16384
VOCAB = 1000000
EMB = 64
HIDDEN = 32

BLKV = 32768
QV = BLKV // 4
NVBLK = (VOCAB + BLKV - 1) // BLKV
NROWS = NVBLK * QV

NC = 2
NS = 16
NW = NC * NS
B_PER_W = BATCH // NW


def _embed_matmul_body(et1_ref, et2_ref, w1a_ref, w1b_ref, h_ref):
    dn = (((0,), (0,)), ((), ()))
    et1 = et1_ref[...].astype(jnp.bfloat16)
    et2 = et2_ref[...].astype(jnp.bfloat16)
    w1a = w1a_ref[...].astype(jnp.bfloat16)
    w1b = w1b_ref[...].astype(jnp.bfloat16)
    h1 = lax.dot_general(et1, w1a, dn, preferred_element_type=jnp.float32)
    h2 = lax.dot_general(et2, w1b, dn, preferred_element_type=jnp.float32)
    def pack(lo, hi):
        ulo = lax.bitcast_convert_type(lo, jnp.uint32)
        uhi = lax.bitcast_convert_type(hi, jnp.uint32)
        return (uhi & jnp.uint32(0xFFFF0000)) | (ulo >> 16)

    w = jnp.concatenate([
        pack(h1[:QV], h1[QV:2 * QV]),
        pack(h2[:QV], h2[QV:2 * QV]),
        pack(h1[2 * QV:3 * QV], h1[3 * QV:]),
        pack(h2[2 * QV:3 * QV], h2[3 * QV:]),
    ], axis=1)
    h_ref[...] = lax.bitcast_convert_type(w, jnp.float32)


def _embed_matmul(ET1, ET2, W1a, W1b):
    return pl.pallas_call(
        _embed_matmul_body,
        grid=(NVBLK,),
        in_specs=[
            pl.BlockSpec((EMB, BLKV), lambda i: (0, i)),
            pl.BlockSpec((EMB, BLKV), lambda i: (0, i)),
            pl.BlockSpec((EMB, HIDDEN), lambda i: (0, 0)),
            pl.BlockSpec((EMB, HIDDEN), lambda i: (0, 0)),
        ],
        out_specs=pl.BlockSpec((QV, 128), lambda i: (i, 0)),
        out_shape=jax.ShapeDtypeStruct((NROWS, 128), jnp.float32),
        compiler_params=pltpu.CompilerParams(
            dimension_semantics=("parallel",),
        ),
    )(ET1, ET2, W1a, W1b)


def _sc_gather_one(H, row):
    mesh = plsc.VectorSubcoreMesh(core_axis_name="c", subcore_axis_name="s")

    @functools.partial(
        pl.kernel,
        mesh=mesh,
        out_type=jax.ShapeDtypeStruct((BATCH, 128), jnp.float32),
        scratch_types=[
            pltpu.VMEM((B_PER_W,), jnp.int32),
            pltpu.VMEM((B_PER_W, 128), jnp.float32),
            pltpu.SemaphoreType.DMA,
        ],
        compiler_params=pltpu.CompilerParams(use_tc_tiling_on_sc=False),
    )
    def k(h_hbm, i_hbm, o_hbm, i_v, r_v, s):
        wid = lax.axis_index("s") * NC + lax.axis_index("c")
        base = wid * B_PER_W
        pltpu.sync_copy(i_hbm.at[pl.ds(base, B_PER_W)], i_v)
        pltpu.async_copy(h_hbm.at[i_v], r_v, s).wait()
        pltpu.sync_copy(r_v, o_hbm.at[pl.ds(base, B_PER_W)])

    return k(H, row)


def _unpack_half(g, half, lane0):
    bits = lax.bitcast_convert_type(g[:, lane0:lane0 + HIDDEN], jnp.uint32)
    lo = lax.bitcast_convert_type(bits << 16, jnp.float32)
    hi = lax.bitcast_convert_type(bits & jnp.uint32(0xFFFF0000), jnp.float32)
    return jnp.where(half > 0, hi, lo)


def _select_quarter(g, sel, lane0):
    a = _unpack_half(g, sel & 1, lane0)
    b = _unpack_half(g, sel & 1, lane0 + 2 * HIDDEN)
    return jnp.where((sel >> 1) > 0, b, a)


def _mlp_body(g1_ref, g2_ref, f1_ref, f2_ref, b1_ref, w2_ref, b2_ref, o_ref):
    h1 = _select_quarter(g1_ref[...], f1_ref[...], 0)
    h2 = _select_quarter(g2_ref[...], f2_ref[...], HIDDEN)
    h = jnp.maximum(h1 + h2 + b1_ref[...], 0.0)
    o = jnp.dot(h, w2_ref[...], preferred_element_type=jnp.float32) + b2_ref[...]
    o_ref[...] = jax.nn.sigmoid(o)


def _tc_mlp(g1, g2, f1, f2, b1, W2, b2):
    BLK = 4096
    return pl.pallas_call(
        _mlp_body,
        grid=(BATCH // BLK,),
        in_specs=[
            pl.BlockSpec((BLK, 128), lambda i: (i, 0)),
            pl.BlockSpec((BLK, 128), lambda i: (i, 0)),
            pl.BlockSpec((BLK, 1), lambda i: (i, 0)),
            pl.BlockSpec((BLK, 1), lambda i: (i, 0)),
            pl.BlockSpec((1, HIDDEN), lambda i: (0, 0)),
            pl.BlockSpec((HIDDEN, 1), lambda i: (0, 0)),
            pl.BlockSpec((1, 1), lambda i: (0, 0)),
        ],
        out_specs=pl.BlockSpec((BLK, 1), lambda i: (i, 0)),
        out_shape=jax.ShapeDtypeStruct((BATCH, 1), jnp.float32),
    )(g1, g2, f1, f2, b1, W2, b2)


_BLKV_BITS = BLKV.bit_length() - 1
_QV_BITS = QV.bit_length() - 1


def _row_half(idx):
    blk = idx >> _BLKV_BITS
    r = idx & (BLKV - 1)
    row = (blk << _QV_BITS) | (r & (QV - 1))
    sel = r >> _QV_BITS
    return row, sel


def kernel(inputs, E1, E2, W1, b1, W2, b2):
    idx1 = inputs[:, 0]
    idx2 = inputs[:, 1]
    ET1 = E1.T
    ET2 = E2.T
    W1a = W1[:EMB]
    W1b = W1[EMB:]
    H = _embed_matmul(ET1, ET2, W1a, W1b)
    row1, half1 = _row_half(idx1)
    row2, half2 = _row_half(idx2)
    g1 = _sc_gather_one(H, row1)
    g2 = _sc_gather_one(H, row2)
    return _tc_mlp(g1, g2, half1.reshape(BATCH, 1), half2.reshape(BATCH, 1),
                   b1.reshape(1, HIDDEN), W2, b2.reshape(1, 1))
